# baseline (speedup 1.0000x reference)
import jax
import jax.numpy as jnp
from jax import lax
from jax.experimental import pallas as pl
from jax.experimental.pallas import tpu as pltpu
from jax.experimental.pallas import tpu_sc as plsc

INSN = 12
NINSN = 10
SEQ = 120
D = 64
NGRP = 3
OUT_PER_ROW = NGRP * NINSN

NC, NS = 2, 16
NW = NC * NS

B = 4096
ROWS_PER_W = B // NW
BBLK = 32
NBLK = ROWS_PER_W // BBLK
GROWS = BBLK * INSN
WIDX = ROWS_PER_W * SEQ


def _body(idx_hbm, table_hbm, pos_hbm, out_hbm,
          idx_v, rows_v0, rows_v1, out_v0, out_v1, pos_v,
          gsem0, gsem1, osem0, osem1):
    wid = lax.axis_index("s") * NC + lax.axis_index("c")
    rows_v = (rows_v0, rows_v1)
    out_v = (out_v0, out_v1)
    gsem = (gsem0, gsem1)
    osem = (osem0, osem1)

    pltpu.sync_copy(pos_hbm, pos_v)
    pltpu.sync_copy(idx_hbm.at[pl.ds(wid * WIDX, WIDX)], idx_v)

    i16 = lax.iota(jnp.int32, 16)
    dh_vec = [(q * 16 + i16) >> 3 for q in range(4)]
    dl_vec = [(q * 16 + i16) & 7 for q in range(4)]
    g_vec = [jnp.full((16,), g, jnp.int32) for g in range(NGRP)]

    def gather_start(p, buf):
        pltpu.async_copy(
            table_hbm.at[idx_v.at[pl.ds(p * GROWS, GROWS)]],
            rows_v[buf], gsem[buf])

    def gather_wait(buf):
        pltpu.make_async_copy(
            table_hbm.at[idx_v.at[pl.ds(0, GROWS)]],
            rows_v[buf], gsem[buf]).wait()

    def out_dst(blk, j):
        return out_hbm.at[pl.ds(3 * j, NGRP), :, wid, :,
                          pl.ds(BBLK * blk, BBLK)]

    gather_start(0, 0)

    @pl.loop(0, NBLK)
    def block(blk):
        p0 = blk * NINSN
        for j in range(NINSN):
            bj = j % 2

            if j < NINSN - 1:
                gather_start(p0 + j + 1, 1 - bj)
            else:
                @pl.when(blk < NBLK - 1)
                def _():
                    gather_start(p0 + NINSN, 1 - bj)

            gather_wait(bj)

            if j >= 2:
                pltpu.make_async_copy(out_v[bj], out_dst(blk, j),
                                      osem[bj]).wait()
            else:
                @pl.when(blk > 0)
                def _():
                    pltpu.make_async_copy(out_v[bj], out_dst(blk, j),
                                          osem[bj]).wait()

            rows = rows_v[bj]
            ob = out_v[bj]
            pos_q = [pos_v[j, pl.ds(q * 16, 16)] for q in range(4)]

            @pl.loop(0, BBLK)
            def brow(b32):
                ro = b32 * INSN
                b_bc = jnp.full((16,), b32, jnp.int32)
                for q in range(4):
                    sl = pl.ds(q * 16, 16)
                    a1 = rows[ro + 0, sl] + rows[ro + 1, sl]
                    a2 = ((rows[ro + 2, sl] + rows[ro + 3, sl])
                          + (rows[ro + 4, sl] + rows[ro + 5, sl])
                          + rows[ro + 6, sl])
                    a3 = ((rows[ro + 7, sl] + rows[ro + 8, sl])
                          + (rows[ro + 9, sl] + rows[ro + 10, sl])
                          + rows[ro + 11, sl])
                    for g, a in ((0, a1), (1, a2), (2, a3)):
                        plsc.store_scatter(
                            ob, [g_vec[g], dh_vec[q], dl_vec[q], b_bc],
                            a + pos_q[q])

            pltpu.async_copy(out_v[bj], out_dst(blk, j), osem[bj])

    for bj in range(2):
        pltpu.make_async_copy(out_v[bj], out_dst(NBLK - 1, NINSN - 2 + bj),
                              osem[bj]).wait()


@jax.jit
def _run(idx_perm, word_table, pos10):
    mesh = plsc.VectorSubcoreMesh(
        core_axis_name="c", subcore_axis_name="s", num_cores=NC, num_subcores=NS)
    k = pl.kernel(
        _body,
        out_type=jax.ShapeDtypeStruct((OUT_PER_ROW, 8, NW, 8, 128),
                                      jnp.float32),
        mesh=mesh,
        scratch_types=[
            pltpu.VMEM((WIDX,), jnp.int32),
            pltpu.VMEM((GROWS, D), jnp.float32),
            pltpu.VMEM((GROWS, D), jnp.float32),
            pltpu.VMEM((NGRP, 8, 8, BBLK), jnp.float32),
            pltpu.VMEM((NGRP, 8, 8, BBLK), jnp.float32),
            pltpu.VMEM((NINSN, D), jnp.float32),
            pltpu.SemaphoreType.DMA,
            pltpu.SemaphoreType.DMA,
            pltpu.SemaphoreType.DMA,
            pltpu.SemaphoreType.DMA,
        ],
        compiler_params=pltpu.CompilerParams(
            use_tc_tiling_on_sc=False, needs_layout_passes=False),
    )
    return k(idx_perm, word_table, pos10)


def kernel(word, word_table, pos_table):
    idx_perm = (word.astype(jnp.int32)
                .reshape(B // BBLK, BBLK, NINSN, INSN)
                .transpose(0, 2, 1, 3)
                .reshape(-1))
    pos10 = lax.slice_in_dim(pos_table, 1, 1 + NINSN, axis=0)
    out5 = _run(idx_perm, word_table, pos10)
    return out5.transpose(2, 4, 0, 1, 3).reshape(B, OUT_PER_ROW, D)
